# baseline (speedup 1.0000x reference)
import functools

import jax
import jax.numpy as jnp
from jax import lax
from jax.experimental import pallas as pl
from jax.experimental.pallas import tpu as pltpu
from jax.experimental.pallas import tpu_sc as plsc

NC = 2
NS = 16
NW = NC * NS
IDX_W = 128
STREAMS = 4
CHUNK = IDX_W * STREAMS
NBUF = 2


def _out_fix(rows, batch, seq, dim, chunks):
    bc = batch // chunks

    def body(i_ref, o_ref):
        o_ref[...] = i_ref[...].reshape(bc, seq, dim)

    return pl.pallas_call(
        body,
        grid=(chunks,),
        in_specs=[pl.BlockSpec((bc * seq, dim), lambda i: (i, 0))],
        out_specs=pl.BlockSpec((bc, seq, dim), lambda i: (i, 0, 0)),
        out_shape=jax.ShapeDtypeStruct((batch, seq, dim), jnp.float32),
    )(rows)


G = 2


@functools.partial(jax.jit, static_argnums=(2, 3, 4))
def _sc_gather(table, ids, batch, seq, dim):
    b = batch * seq
    bpt = batch // NW
    n_groups = bpt // G
    seq_lo = min(IDX_W, seq)
    seq_hi = seq - seq_lo

    mesh = plsc.VectorSubcoreMesh(core_axis_name="c", subcore_axis_name="s")

    @functools.partial(
        pl.kernel,
        mesh=mesh,
        out_type=jax.ShapeDtypeStruct((b, dim), jnp.float32),
        scratch_types=[
            pltpu.VMEM((bpt, seq), jnp.int32),
            pltpu.VMEM((NBUF, G * seq, dim), jnp.float32),
            pltpu.SemaphoreType.DMA((NBUF,)),
            pltpu.SemaphoreType.DMA((NBUF,)),
            pltpu.SemaphoreType.DMA,
        ],
        compiler_params=pltpu.CompilerParams(use_tc_tiling_on_sc=False),
    )
    def k(table_hbm, ids_hbm, out_hbm, idx_v, rows_v, gsem, wsem, isem):
        wid = lax.axis_index("s") * NC + lax.axis_index("c")
        base = wid * bpt

        pltpu.async_copy(ids_hbm.at[pl.ds(base, bpt)], idx_v, isem).wait()

        def fire_gathers(grp, buf):
            copies = []
            for g in range(G):
                row = grp * G + g
                copies.append(
                    pltpu.async_copy(
                        table_hbm.at[idx_v.at[row, pl.ds(0, seq_lo)]],
                        rows_v.at[buf, pl.ds(g * seq, seq_lo)],
                        gsem.at[buf],
                    )
                )
                if seq_hi:
                    copies.append(
                        pltpu.async_copy(
                            table_hbm.at[idx_v.at[row, pl.ds(seq_lo, seq_hi)]],
                            rows_v.at[buf, pl.ds(g * seq + seq_lo, seq_hi)],
                            gsem.at[buf],
                        )
                    )
            return copies

        def fire_writeout(grp, buf):
            return pltpu.async_copy(
                rows_v.at[buf],
                out_hbm.at[pl.ds((base + grp * G) * seq, G * seq)],
                wsem.at[buf],
            )

        @pl.loop(0, n_groups, step=NBUF)
        def _(j):
            gathers = [fire_gathers(j + bf, bf) for bf in range(NBUF)]
            writes = []
            for bf in range(NBUF):
                for gth in gathers[bf]:
                    gth.wait()
                writes.append(fire_writeout(j + bf, bf))
            for w in writes:
                w.wait()

    return k(table, ids)


def kernel(input_ids, embedding_matrix):
    batch, seq = input_ids.shape
    dim = embedding_matrix.shape[1]
    rows = _sc_gather(embedding_matrix, input_ids, batch, seq, dim)
    return _out_fix(rows, batch, seq, dim, chunks=64)


# speedup vs baseline: 1.2971x; 1.2971x over previous
import functools

import jax
import jax.numpy as jnp
from jax import lax
from jax.experimental import pallas as pl
from jax.experimental.pallas import tpu as pltpu
from jax.experimental.pallas import tpu_sc as plsc

NC = 2
NS = 16
NW = NC * NS
IDX_W = 128
STREAMS = 4
CHUNK = IDX_W * STREAMS
NBUF = 2


@functools.partial(jax.jit, static_argnums=(2, 3))
def _sc_gather(table, idx2d, b, dim):
    b_per_w = b // NW
    rows_per_w = b_per_w // IDX_W
    n_chunks = b_per_w // CHUNK

    mesh = plsc.VectorSubcoreMesh(core_axis_name="c", subcore_axis_name="s")

    @functools.partial(
        pl.kernel,
        mesh=mesh,
        out_type=jax.ShapeDtypeStruct((b, dim), jnp.float32),
        scratch_types=[
            pltpu.VMEM((rows_per_w, IDX_W), jnp.int32),
            pltpu.VMEM((NBUF, CHUNK, dim), jnp.float32),
            pltpu.SemaphoreType.DMA((NBUF,)),
            pltpu.SemaphoreType.DMA((NBUF,)),
            pltpu.SemaphoreType.DMA,
        ],
        compiler_params=pltpu.CompilerParams(use_tc_tiling_on_sc=False),
    )
    def k(table_hbm, idx_hbm, out_hbm, idx_v, rows_v, gsem, wsem, isem):
        wid = lax.axis_index("s") * NC + lax.axis_index("c")
        base = wid * b_per_w

        pltpu.async_copy(
            idx_hbm.at[pl.ds(pl.multiple_of(base // IDX_W, 8), rows_per_w)],
            idx_v,
            isem,
        ).wait()

        def fire_gathers(c, buf):
            copies = []
            for i in range(STREAMS):
                copies.append(
                    pltpu.async_copy(
                        table_hbm.at[idx_v.at[c * STREAMS + i]],
                        rows_v.at[buf, pl.ds(i * IDX_W, IDX_W)],
                        gsem.at[buf],
                    )
                )
            return copies

        def fire_writeout(c, buf):
            off = pl.multiple_of(base + c * CHUNK, CHUNK)
            return pltpu.async_copy(
                rows_v.at[buf], out_hbm.at[pl.ds(off, CHUNK)], wsem.at[buf]
            )

        @pl.loop(0, n_chunks, step=NBUF)
        def _(j):
            gathers = [fire_gathers(j + bf, bf) for bf in range(NBUF)]
            writes = []
            for bf in range(NBUF):
                for g in gathers[bf]:
                    g.wait()
                writes.append(fire_writeout(j + bf, bf))
            for w in writes:
                w.wait()

    return k(table, idx2d)


def kernel(input_ids, embedding_matrix):
    batch, seq = input_ids.shape
    n_emb, dim = embedding_matrix.shape
    b = batch * seq
    idx2d = lax.rem(
        input_ids.astype(jnp.int32).reshape(b // IDX_W, IDX_W),
        jnp.int32(n_emb),
    )
    out = _sc_gather(embedding_matrix, idx2d, b, dim)
    return out.reshape(batch, seq, dim)
